# baseline (speedup 1.0000x reference)
import jax
import jax.numpy as jnp
from jax.experimental import pallas as pl

_BM = 2000


def _fused_linear_kernel(x_ref, wct_ref, bc_ref, wbt_ref, bb_ref, s_ref, d_ref):
    x = x_ref[...]
    s_ref[...] = (
        jnp.dot(x, wct_ref[...], preferred_element_type=jnp.float32) + bc_ref[...]
    )
    d_ref[...] = (
        jnp.dot(x, wbt_ref[...], preferred_element_type=jnp.float32) + bb_ref[...]
    )


def kernel(x, Wc, bc, Wb, bb):
    if x.ndim > 2:
        x = x.reshape(x.shape[0], -1)
    n, d = x.shape
    c1 = Wc.shape[0]
    c2 = Wb.shape[0]
    bm = _BM if n % _BM == 0 else n
    wct = Wc.T
    wbt = Wb.T
    bc2 = bc.reshape(1, c1)
    bb2 = bb.reshape(1, c2)
    scores, deltas = pl.pallas_call(
        _fused_linear_kernel,
        grid=(n // bm,),
        in_specs=[
            pl.BlockSpec((bm, d), lambda i: (i, 0)),
            pl.BlockSpec((d, c1), lambda i: (0, 0)),
            pl.BlockSpec((1, c1), lambda i: (0, 0)),
            pl.BlockSpec((d, c2), lambda i: (0, 0)),
            pl.BlockSpec((1, c2), lambda i: (0, 0)),
        ],
        out_specs=[
            pl.BlockSpec((bm, c1), lambda i: (i, 0)),
            pl.BlockSpec((bm, c2), lambda i: (i, 0)),
        ],
        out_shape=[
            jax.ShapeDtypeStruct((n, c1), x.dtype),
            jax.ShapeDtypeStruct((n, c2), x.dtype),
        ],
    )(x, wct, bc2, wbt, bb2)
    return (scores, deltas)
